# baseline (speedup 1.0000x reference)
import functools

import jax
import jax.numpy as jnp
from jax import lax
from jax.experimental import pallas as pl
from jax.experimental.pallas import tpu as pltpu
from jax.experimental.pallas import tpu_sc as plsc

N = 518400
B = 65536
C = 49
CW = 56
OPACITY_COL = 13

NC = 2
NS = 16
NW = NC * NS
BPW = B // NW
CHUNK = 128
NCHUNK = BPW // CHUNK

PACK_BS = 3840


def _t_body(o_ref, t_ref):
    x = o_ref[...]
    p = jnp.clip(jax.nn.sigmoid(x) * 0.5, 1e-4, 1.0 - 1e-4)
    t_ref[...] = jnp.log(p) - jnp.log1p(-p)


_t_call = pl.pallas_call(
    _t_body,
    out_shape=jax.ShapeDtypeStruct((N // 128, 128), jnp.float32),
    in_specs=[pl.BlockSpec((N // 128, 128), lambda: (0, 0))],
    out_specs=pl.BlockSpec((N // 128, 128), lambda: (0, 0)),
)


def _pack_body(uv, depth, xyz, quat, scale, opac, rgb, latent, out):
    row = jnp.concatenate(
        [uv[...], depth[...], xyz[...], quat[...], scale[...], opac[...],
         rgb[...], latent[...]], axis=1)
    out[...] = jnp.pad(row, ((0, 0), (0, CW - C)))


_pack_call = pl.pallas_call(
    _pack_body,
    grid=(N // PACK_BS,),
    in_specs=[
        pl.BlockSpec((PACK_BS, w), lambda i: (i, 0))
        for w in (2, 1, 3, 4, 3, 1, 3, 32)
    ],
    out_specs=pl.BlockSpec((PACK_BS, CW), lambda i: (i, 0)),
    out_shape=jax.ShapeDtypeStruct((N + B, CW), jnp.float32),
    compiler_params=pltpu.CompilerParams(dimension_semantics=("parallel",)),
)


@functools.partial(
    pl.kernel,
    out_type=(),
    mesh=plsc.VectorSubcoreMesh(core_axis_name="c", subcore_axis_name="s"),
    compiler_params=pltpu.CompilerParams(needs_layout_passes=False,
                                         use_tc_tiling_on_sc=False),
    scratch_types=[
        pltpu.VMEM((NCHUNK, CHUNK), jnp.int32),
        pltpu.VMEM((CHUNK, CW), jnp.float32),
        pltpu.VMEM((CHUNK,), jnp.float32),
        pltpu.SemaphoreType.DMA,
    ],
)
def _clone_kernel(big_ref, t_hbm, idx_hbm, idx_v, rows_v, tv_v, sem):
    wid = lax.axis_index("s") * NC + lax.axis_index("c")
    pltpu.sync_copy(idx_hbm.at[wid], idx_v)
    cid = jnp.full((16,), OPACITY_COL, jnp.int32)
    for j in range(NCHUNK):
        ij = idx_v.at[j]
        pltpu.async_copy(big_ref.at[ij], rows_v, sem).wait()
        pltpu.async_copy(t_hbm.at[ij], tv_v, sem).wait()
        for v in range(CHUNK // 16):
            rid = lax.iota(jnp.int32, 16) + (v * 16)
            plsc.store_scatter(rows_v, [rid, cid], tv_v[pl.ds(v * 16, 16)])
        row0 = N + wid * BPW + j * CHUNK
        pltpu.sync_copy(rows_v, big_ref.at[pl.ds(row0, CHUNK)])
        pltpu.async_copy(rows_v, big_ref.at[ij], sem).wait()


def kernel(uv, depth_raw, xyz_offset, quat_raw, log_scale, opacity_logit,
           rgb_logit, latent, indices):
    t = _t_call(opacity_logit.reshape(N // 128, 128)).reshape(N)
    big0 = _pack_call(uv, depth_raw, xyz_offset, quat_raw, log_scale,
                      opacity_logit, rgb_logit, latent)
    ref = jax.new_ref(big0)
    _clone_kernel(ref, t, indices.reshape(NW, NCHUNK, CHUNK))
    return jax.freeze(ref)[:, :C]
